# baseline (speedup 1.0000x reference)
import functools

import jax
import jax.numpy as jnp
from jax import lax
from jax.experimental import pallas as pl
from jax.experimental.pallas import tpu as pltpu
from jax.experimental.pallas import tpu_sc as plsc

NC = 2
NS = 16
NW = NC * NS
CHUNK = 128

N_PAD = 10240
DUMMY = 10200
BM = 2048


def _mesh():
  return plsc.VectorSubcoreMesh(
      core_axis_name="c", subcore_axis_name="s", num_cores=NC, num_subcores=NS
  )


def _make_deg(n_chunks):
  rows_per_tile = N_PAD // NS

  @functools.partial(
      pl.kernel,
      out_type=jax.ShapeDtypeStruct((NC, N_PAD), jnp.float32),
      mesh=_mesh(),
      scratch_types=[
          pltpu.VMEM((n_chunks, CHUNK), jnp.int32),
          pltpu.VMEM((CHUNK,), jnp.float32),
          pltpu.VMEM_SHARED((N_PAD,), jnp.float32),
          pltpu.SemaphoreType.DMA,
      ],
  )
  def deg_kernel(dst_hbm, zero_hbm, out_hbm, dst_v, ones_v, acc_sh, sem):
    cid = lax.axis_index("c")
    sid = lax.axis_index("s")
    wid = cid * NS + sid
    r0 = sid * rows_per_tile
    pltpu.sync_copy(zero_hbm.at[pl.ds(r0, rows_per_tile)],
                    acc_sh.at[pl.ds(r0, rows_per_tile)])
    pltpu.sync_copy(dst_hbm.at[wid], dst_v)
    for i in range(CHUNK // 16):
      ones_v[pl.ds(i * 16, 16)] = jnp.full((16,), 1.0, jnp.float32)
    plsc.subcore_barrier()
    kk = 4

    def body(g, carry):
      for k in range(kk):
        pltpu.async_copy(ones_v, acc_sh.at[dst_v.at[g * kk + k]], sem,
                         add=True)
      for k in range(kk):
        pltpu.make_async_copy(ones_v, acc_sh.at[dst_v.at[g * kk + k]],
                              sem).wait()
      return carry

    assert n_chunks % kk == 0
    lax.fori_loop(0, n_chunks // kk, body, 0)
    plsc.subcore_barrier()
    pltpu.sync_copy(acc_sh.at[pl.ds(r0, rows_per_tile)],
                    out_hbm.at[cid, pl.ds(r0, rows_per_tile)])

  return deg_kernel


def _agg_loop(p_tbls, src_t, dst_t, idxv, rows_v, acc_sh, gsem, ssem, isem,
              nbuf, n_super):
  pltpu.sync_copy(src_t.at[0], idxv.at[0, 0])
  pltpu.sync_copy(dst_t.at[0], idxv.at[0, 1])
  for b in range(nbuf):
    pltpu.async_copy(p_tbls[b].at[idxv.at[0, 0, b]], rows_v.at[b], gsem.at[b])
  if n_super > 1:
    pltpu.async_copy(src_t.at[1], idxv.at[1, 0], isem)
    pltpu.async_copy(dst_t.at[1], idxv.at[1, 1], isem)

  def body(g, carry):
    r = lax.rem(g, 2)
    for b in range(nbuf):
      pltpu.make_async_copy(
          p_tbls[b].at[idxv.at[r, 0, b]], rows_v.at[b], gsem.at[b]).wait()
      pltpu.async_copy(
          rows_v.at[b], acc_sh.at[idxv.at[r, 1, b]], ssem.at[b], add=True)

    @pl.when(g < n_super - 1)
    def _():
      pltpu.make_async_copy(src_t.at[g + 1], idxv.at[1 - r, 0], isem).wait()
      pltpu.make_async_copy(dst_t.at[g + 1], idxv.at[1 - r, 1], isem).wait()

    for b in range(nbuf):
      pltpu.make_async_copy(
          rows_v.at[b], acc_sh.at[idxv.at[r, 1, b]], ssem.at[b]).wait()

      @pl.when(g < n_super - 1)
      def _():
        pltpu.async_copy(
            p_tbls[b].at[idxv.at[1 - r, 0, b]], rows_v.at[b], gsem.at[b])

    @pl.when(g < n_super - 2)
    def _():
      pltpu.async_copy(src_t.at[g + 2], idxv.at[r, 0], isem)
      pltpu.async_copy(dst_t.at[g + 2], idxv.at[r, 1], isem)

    return carry

  lax.fori_loop(0, n_super, body, 0)


def _make_agg_cols(n_super, half, nbuf):
  rows_per_tile = N_PAD // NS

  @functools.partial(
      pl.kernel,
      out_type=jax.ShapeDtypeStruct((N_PAD, 2 * half), jnp.float32),
      mesh=_mesh(),
      compiler_params=pltpu.CompilerParams(use_tc_tiling_on_sc=False),
      scratch_types=[
          pltpu.VMEM((2, 2, nbuf, CHUNK), jnp.int32),
          pltpu.VMEM((nbuf, CHUNK, half), jnp.float32),
          pltpu.VMEM_SHARED((N_PAD, half), jnp.float32),
          pltpu.VMEM_SHARED((N_PAD, half), jnp.float32),
          pltpu.SemaphoreType.DMA((nbuf,)),
          pltpu.SemaphoreType.DMA((nbuf,)),
          pltpu.SemaphoreType.DMA,
      ],
  )
  def agg_kernel(p_hbm, src_hbm, dst_hbm, out_hbm,
                 idxv, rows_v, p_sh, acc_sh, gsem, ssem, isem):
    cid = lax.axis_index("c")
    sid = lax.axis_index("s")
    r0 = sid * rows_per_tile
    c0 = cid * half
    pltpu.sync_copy(p_hbm.at[pl.ds(r0, rows_per_tile), pl.ds(c0, half)],
                    p_sh.at[pl.ds(r0, rows_per_tile)])
    pltpu.sync_copy(p_hbm.at[pl.ds(r0, rows_per_tile), pl.ds(c0, half)],
                    acc_sh.at[pl.ds(r0, rows_per_tile)])
    plsc.subcore_barrier()
    _agg_loop([p_sh] * nbuf, src_hbm.at[sid], dst_hbm.at[sid], idxv, rows_v,
              acc_sh, gsem, ssem, isem, nbuf, n_super)
    plsc.subcore_barrier()
    pltpu.sync_copy(acc_sh.at[pl.ds(r0, rows_per_tile)],
                    out_hbm.at[pl.ds(r0, rows_per_tile), pl.ds(c0, half)])

  return agg_kernel


def _make_agg_edges(n_super, width, nbuf):
  rows_per_tile = N_PAD // NS

  @functools.partial(
      pl.kernel,
      out_type=jax.ShapeDtypeStruct((NC, N_PAD, width), jnp.float32),
      mesh=_mesh(),
      compiler_params=pltpu.CompilerParams(use_tc_tiling_on_sc=False),
      scratch_types=[
          pltpu.VMEM((2, 2, nbuf, CHUNK), jnp.int32),
          pltpu.VMEM((nbuf, CHUNK, width), jnp.float32),
          pltpu.VMEM_SHARED((N_PAD, width), jnp.float32),
          pltpu.VMEM_SHARED((N_PAD, width), jnp.float32),
          pltpu.SemaphoreType.DMA((nbuf,)),
          pltpu.SemaphoreType.DMA((nbuf,)),
          pltpu.SemaphoreType.DMA,
      ],
  )
  def agg_kernel(p_hbm, src_hbm, dst_hbm, zero_hbm, out_hbm,
                 idxv, rows_v, p_sh, acc_sh, gsem, ssem, isem):
    cid = lax.axis_index("c")
    sid = lax.axis_index("s")
    wid = cid * NS + sid
    r0 = sid * rows_per_tile
    pltpu.sync_copy(p_hbm.at[pl.ds(r0, rows_per_tile)],
                    p_sh.at[pl.ds(r0, rows_per_tile)])

    @pl.when(cid == 0)
    def _():
      pltpu.sync_copy(p_hbm.at[pl.ds(r0, rows_per_tile)],
                      acc_sh.at[pl.ds(r0, rows_per_tile)])

    @pl.when(cid != 0)
    def _():
      pltpu.sync_copy(zero_hbm.at[pl.ds(r0, rows_per_tile)],
                      acc_sh.at[pl.ds(r0, rows_per_tile)])

    plsc.subcore_barrier()
    tbls = [p_sh, p_hbm, p_sh, p_sh, p_hbm][:nbuf]
    _agg_loop(tbls, src_hbm.at[wid], dst_hbm.at[wid], idxv, rows_v,
              acc_sh, gsem, ssem, isem, nbuf, n_super)
    plsc.subcore_barrier()
    pltpu.sync_copy(acc_sh.at[pl.ds(r0, rows_per_tile)],
                    out_hbm.at[cid, pl.ds(r0, rows_per_tile)])

  return agg_kernel


def _tc_first(x, w1, degp):
  hidden = w1.shape[1]

  def body(x_ref, w_ref, deg_ref, p_ref, dinv_ref):
    deg = deg_ref[0] + deg_ref[1] + 1.0
    dinv = lax.rsqrt(deg)
    h = jnp.dot(x_ref[...], w_ref[...], preferred_element_type=jnp.float32)
    p_ref[...] = h * dinv
    dinv_ref[...] = dinv

  grid = (N_PAD // BM,)
  return pl.pallas_call(
      body,
      grid=grid,
      in_specs=[
          pl.BlockSpec((BM, x.shape[1]), lambda i: (i, 0)),
          pl.BlockSpec(w1.shape, lambda i: (0, 0)),
          pl.BlockSpec((NC, BM, 1), lambda i: (0, i, 0)),
      ],
      out_specs=[
          pl.BlockSpec((BM, hidden), lambda i: (i, 0)),
          pl.BlockSpec((BM, 1), lambda i: (i, 0)),
      ],
      out_shape=[
          jax.ShapeDtypeStruct((N_PAD, hidden), jnp.float32),
          jax.ShapeDtypeStruct((N_PAD, 1), jnp.float32),
      ],
  )(x, w1, degp)


def _tc_mid(agg1, dinv, b1, w2p):
  width = w2p.shape[1]
  hidden = agg1.shape[1]

  def body(agg_ref, dinv_ref, b_ref, w_ref, p2_ref):
    s = agg_ref[...] * dinv_ref[...]
    h = jnp.maximum(s + b_ref[...], 0.0)
    p2_ref[...] = jnp.dot(h, w_ref[...],
                          preferred_element_type=jnp.float32) * dinv_ref[...]

  grid = (N_PAD // BM,)
  return pl.pallas_call(
      body,
      grid=grid,
      in_specs=[
          pl.BlockSpec((BM, hidden), lambda i: (i, 0)),
          pl.BlockSpec((BM, 1), lambda i: (i, 0)),
          pl.BlockSpec((1, hidden), lambda i: (0, 0)),
          pl.BlockSpec(w2p.shape, lambda i: (0, 0)),
      ],
      out_specs=pl.BlockSpec((BM, width), lambda i: (i, 0)),
      out_shape=jax.ShapeDtypeStruct((N_PAD, width), jnp.float32),
  )(agg1, dinv, b1, w2p)


def _tc_last(aggp, dinv, b2p, n, n_classes):
  width = aggp.shape[2]
  bm = 2000 if n % 2000 == 0 else 2048
  assert n % bm == 0

  def body(agg_ref, dinv_ref, b_ref, out_ref):
    s = (agg_ref[0] + agg_ref[1]) * dinv_ref[...]
    out_ref[...] = (s + b_ref[...])[:, :n_classes]

  grid = (n // bm,)
  return pl.pallas_call(
      body,
      grid=grid,
      in_specs=[
          pl.BlockSpec((NC, bm, width), lambda i: (0, i, 0)),
          pl.BlockSpec((bm, 1), lambda i: (i, 0)),
          pl.BlockSpec((1, width), lambda i: (0, 0)),
      ],
      out_specs=pl.BlockSpec((bm, n_classes), lambda i: (i, 0)),
      out_shape=jax.ShapeDtypeStruct((n, n_classes), jnp.float32),
  )(aggp, dinv, b2p)


def kernel(x, edge_index, W1, b1, W2, b2):
  n, d_feat = x.shape
  hidden = W1.shape[1]
  n_classes = W2.shape[1]
  e = edge_index.shape[1]

  nbuf1, nbuf2 = 5, 5
  per = NW * CHUNK
  n2 = -(-(-(-e // per)) // 10) * 10
  e_pad = n2 * per
  n1 = 2 * n2
  ns1, ns2 = n1 // nbuf1, n2 // nbuf2
  ei = jnp.pad(edge_index.astype(jnp.int32), ((0, 0), (0, e_pad - e)),
               constant_values=DUMMY)
  src_f = ei[0]
  dst_f = ei[1]
  src_c = src_f.reshape(NS, ns1, nbuf1, CHUNK)
  dst_c = dst_f.reshape(NS, ns1, nbuf1, CHUNK)
  src_p = src_f.reshape(NW, ns2, nbuf2, CHUNK)
  dst_p = dst_f.reshape(NW, ns2, nbuf2, CHUNK)
  dst_d = dst_f.reshape(NW, n2, CHUNK)

  x_p = jnp.pad(x, ((0, N_PAD - n), (0, 0)))
  w2_width = -(-n_classes // 16) * 16
  w2_p = jnp.pad(W2, ((0, 0), (0, w2_width - n_classes)))
  b1_r = b1.reshape(1, hidden)
  b2_r = jnp.pad(b2, (0, w2_width - n_classes)).reshape(1, w2_width)

  zeros1 = jnp.zeros((N_PAD,), jnp.float32)
  zeros_w = jnp.zeros((N_PAD, w2_width), jnp.float32)

  degp = _make_deg(n2)(dst_d, zeros1)
  p1, dinv = _tc_first(x_p, W1, degp[..., None])
  agg1 = _make_agg_cols(ns1, hidden // 2, nbuf1)(p1, src_c, dst_c)
  p2 = _tc_mid(agg1, dinv, b1_r, w2_p)
  agg2 = _make_agg_edges(ns2, w2_width, nbuf2)(p2, src_p, dst_p, zeros_w)
  if n % 2000 == 0:
    return _tc_last(agg2, dinv, b2_r, n, n_classes)
  out = _tc_last(agg2, dinv, b2_r, N_PAD, w2_width)
  return out[:n, :n_classes]


# speedup vs baseline: 1.0983x; 1.0983x over previous
import functools

import jax
import jax.numpy as jnp
from jax import lax
from jax.experimental import pallas as pl
from jax.experimental.pallas import tpu as pltpu
from jax.experimental.pallas import tpu_sc as plsc

NC = 2
NS = 16
NW = NC * NS
CHUNK = 128

N_PAD = 10240
DUMMY = 10200
BM = 2048


def _mesh():
  return plsc.VectorSubcoreMesh(
      core_axis_name="c", subcore_axis_name="s", num_cores=NC, num_subcores=NS
  )


def _make_deg(n_chunks):
  rows_per_tile = N_PAD // NS

  @functools.partial(
      pl.kernel,
      out_type=jax.ShapeDtypeStruct((NC, N_PAD), jnp.float32),
      mesh=_mesh(),
      scratch_types=[
          pltpu.VMEM((n_chunks, CHUNK), jnp.int32),
          pltpu.VMEM((CHUNK,), jnp.float32),
          pltpu.VMEM_SHARED((N_PAD,), jnp.float32),
          pltpu.SemaphoreType.DMA,
      ],
  )
  def deg_kernel(dst_hbm, zero_hbm, out_hbm, dst_v, ones_v, acc_sh, sem):
    cid = lax.axis_index("c")
    sid = lax.axis_index("s")
    wid = cid * NS + sid
    r0 = sid * rows_per_tile
    pltpu.sync_copy(zero_hbm.at[pl.ds(r0, rows_per_tile)],
                    acc_sh.at[pl.ds(r0, rows_per_tile)])
    pltpu.sync_copy(dst_hbm.at[wid], dst_v)
    for i in range(CHUNK // 16):
      ones_v[pl.ds(i * 16, 16)] = jnp.full((16,), 1.0, jnp.float32)
    plsc.subcore_barrier()
    kk = 4

    def body(g, carry):
      for k in range(kk):
        pltpu.async_copy(ones_v, acc_sh.at[dst_v.at[g * kk + k]], sem,
                         add=True)
      for k in range(kk):
        pltpu.make_async_copy(ones_v, acc_sh.at[dst_v.at[g * kk + k]],
                              sem).wait()
      return carry

    assert n_chunks % kk == 0
    lax.fori_loop(0, n_chunks // kk, body, 0)
    plsc.subcore_barrier()
    pltpu.sync_copy(acc_sh.at[pl.ds(r0, rows_per_tile)],
                    out_hbm.at[cid, pl.ds(r0, rows_per_tile)])

  return deg_kernel


def _agg_loop(p_tbls, src_t, dst_t, idxv, rows_v, acc_sh, gsem, ssem, isem,
              nbuf, n_super):
  pltpu.sync_copy(src_t.at[0], idxv.at[0, 0])
  pltpu.sync_copy(dst_t.at[0], idxv.at[0, 1])
  for b in range(nbuf):
    pltpu.async_copy(p_tbls[b].at[idxv.at[0, 0, b]], rows_v.at[b], gsem.at[b])
  if n_super > 1:
    pltpu.async_copy(src_t.at[1], idxv.at[1, 0], isem)
    pltpu.async_copy(dst_t.at[1], idxv.at[1, 1], isem)

  def body(g, carry):
    r = lax.rem(g, 2)
    for b in range(nbuf):
      pltpu.make_async_copy(
          p_tbls[b].at[idxv.at[r, 0, b]], rows_v.at[b], gsem.at[b]).wait()
      pltpu.async_copy(
          rows_v.at[b], acc_sh.at[idxv.at[r, 1, b]], ssem.at[b], add=True)

    @pl.when(g < n_super - 1)
    def _():
      pltpu.make_async_copy(src_t.at[g + 1], idxv.at[1 - r, 0], isem).wait()
      pltpu.make_async_copy(dst_t.at[g + 1], idxv.at[1 - r, 1], isem).wait()

    for b in range(nbuf):
      pltpu.make_async_copy(
          rows_v.at[b], acc_sh.at[idxv.at[r, 1, b]], ssem.at[b]).wait()

      @pl.when(g < n_super - 1)
      def _():
        pltpu.async_copy(
            p_tbls[b].at[idxv.at[1 - r, 0, b]], rows_v.at[b], gsem.at[b])

    @pl.when(g < n_super - 2)
    def _():
      pltpu.async_copy(src_t.at[g + 2], idxv.at[r, 0], isem)
      pltpu.async_copy(dst_t.at[g + 2], idxv.at[r, 1], isem)

    return carry

  lax.fori_loop(0, n_super, body, 0)


def _make_agg_cols(n_super, half, nbuf):
  rows_per_tile = N_PAD // NS

  @functools.partial(
      pl.kernel,
      out_type=jax.ShapeDtypeStruct((N_PAD, 2 * half), jnp.float32),
      mesh=_mesh(),
      compiler_params=pltpu.CompilerParams(use_tc_tiling_on_sc=False),
      scratch_types=[
          pltpu.VMEM((2, 2, nbuf, CHUNK), jnp.int32),
          pltpu.VMEM((nbuf, CHUNK, half), jnp.float32),
          pltpu.VMEM_SHARED((N_PAD, half), jnp.float32),
          pltpu.VMEM_SHARED((N_PAD, half), jnp.float32),
          pltpu.SemaphoreType.DMA((nbuf,)),
          pltpu.SemaphoreType.DMA((nbuf,)),
          pltpu.SemaphoreType.DMA,
      ],
  )
  def agg_kernel(p_hbm, src_hbm, dst_hbm, out_hbm,
                 idxv, rows_v, p_sh, acc_sh, gsem, ssem, isem):
    cid = lax.axis_index("c")
    sid = lax.axis_index("s")
    r0 = sid * rows_per_tile
    c0 = cid * half
    pltpu.sync_copy(p_hbm.at[pl.ds(r0, rows_per_tile), pl.ds(c0, half)],
                    p_sh.at[pl.ds(r0, rows_per_tile)])
    pltpu.sync_copy(p_hbm.at[pl.ds(r0, rows_per_tile), pl.ds(c0, half)],
                    acc_sh.at[pl.ds(r0, rows_per_tile)])
    plsc.subcore_barrier()
    _agg_loop([p_sh] * nbuf, src_hbm.at[sid], dst_hbm.at[sid], idxv, rows_v,
              acc_sh, gsem, ssem, isem, nbuf, n_super)
    plsc.subcore_barrier()
    pltpu.sync_copy(acc_sh.at[pl.ds(r0, rows_per_tile)],
                    out_hbm.at[pl.ds(r0, rows_per_tile), pl.ds(c0, half)])

  return agg_kernel


def _make_agg_edges(n_super, width, nbuf):
  rows_per_tile = N_PAD // NS

  @functools.partial(
      pl.kernel,
      out_type=jax.ShapeDtypeStruct((NC, N_PAD, width), jnp.float32),
      mesh=_mesh(),
      compiler_params=pltpu.CompilerParams(use_tc_tiling_on_sc=False),
      scratch_types=[
          pltpu.VMEM((2, 2, nbuf, CHUNK), jnp.int32),
          pltpu.VMEM((nbuf, CHUNK, width), jnp.float32),
          pltpu.VMEM_SHARED((N_PAD, width), jnp.float32),
          pltpu.VMEM_SHARED((N_PAD, width), jnp.float32),
          pltpu.SemaphoreType.DMA((nbuf,)),
          pltpu.SemaphoreType.DMA((nbuf,)),
          pltpu.SemaphoreType.DMA,
      ],
  )
  def agg_kernel(p_hbm, src_hbm, dst_hbm, zero_hbm, out_hbm,
                 idxv, rows_v, p_sh, acc_sh, gsem, ssem, isem):
    cid = lax.axis_index("c")
    sid = lax.axis_index("s")
    wid = cid * NS + sid
    r0 = sid * rows_per_tile
    pltpu.sync_copy(p_hbm.at[pl.ds(r0, rows_per_tile)],
                    p_sh.at[pl.ds(r0, rows_per_tile)])

    @pl.when(cid == 0)
    def _():
      pltpu.sync_copy(p_hbm.at[pl.ds(r0, rows_per_tile)],
                      acc_sh.at[pl.ds(r0, rows_per_tile)])

    @pl.when(cid != 0)
    def _():
      pltpu.sync_copy(zero_hbm.at[pl.ds(r0, rows_per_tile)],
                      acc_sh.at[pl.ds(r0, rows_per_tile)])

    plsc.subcore_barrier()
    _agg_loop([p_sh] * nbuf, src_hbm.at[wid], dst_hbm.at[wid], idxv, rows_v,
              acc_sh, gsem, ssem, isem, nbuf, n_super)
    plsc.subcore_barrier()
    pltpu.sync_copy(acc_sh.at[pl.ds(r0, rows_per_tile)],
                    out_hbm.at[cid, pl.ds(r0, rows_per_tile)])

  return agg_kernel


def _tc_first(x, w1, degp):
  hidden = w1.shape[1]

  def body(x_ref, w_ref, deg_ref, p_ref, dinv_ref):
    deg = deg_ref[0] + deg_ref[1] + 1.0
    dinv = lax.rsqrt(deg)
    h = jnp.dot(x_ref[...], w_ref[...], preferred_element_type=jnp.float32)
    p_ref[...] = h * dinv
    dinv_ref[...] = dinv

  grid = (N_PAD // BM,)
  return pl.pallas_call(
      body,
      grid=grid,
      in_specs=[
          pl.BlockSpec((BM, x.shape[1]), lambda i: (i, 0)),
          pl.BlockSpec(w1.shape, lambda i: (0, 0)),
          pl.BlockSpec((NC, BM, 1), lambda i: (0, i, 0)),
      ],
      out_specs=[
          pl.BlockSpec((BM, hidden), lambda i: (i, 0)),
          pl.BlockSpec((BM, 1), lambda i: (i, 0)),
      ],
      out_shape=[
          jax.ShapeDtypeStruct((N_PAD, hidden), jnp.float32),
          jax.ShapeDtypeStruct((N_PAD, 1), jnp.float32),
      ],
  )(x, w1, degp)


def _tc_mid(agg1, dinv, b1, w2p):
  width = w2p.shape[1]
  hidden = agg1.shape[1]

  def body(agg_ref, dinv_ref, b_ref, w_ref, p2_ref):
    s = agg_ref[...] * dinv_ref[...]
    h = jnp.maximum(s + b_ref[...], 0.0)
    p2_ref[...] = jnp.dot(h, w_ref[...],
                          preferred_element_type=jnp.float32) * dinv_ref[...]

  grid = (N_PAD // BM,)
  return pl.pallas_call(
      body,
      grid=grid,
      in_specs=[
          pl.BlockSpec((BM, hidden), lambda i: (i, 0)),
          pl.BlockSpec((BM, 1), lambda i: (i, 0)),
          pl.BlockSpec((1, hidden), lambda i: (0, 0)),
          pl.BlockSpec(w2p.shape, lambda i: (0, 0)),
      ],
      out_specs=pl.BlockSpec((BM, width), lambda i: (i, 0)),
      out_shape=jax.ShapeDtypeStruct((N_PAD, width), jnp.float32),
  )(agg1, dinv, b1, w2p)


def _tc_last(aggp, dinv, b2p, n, n_classes):
  width = aggp.shape[2]
  bm = 2000 if n % 2000 == 0 else 2048
  assert n % bm == 0

  def body(agg_ref, dinv_ref, b_ref, out_ref):
    s = (agg_ref[0] + agg_ref[1]) * dinv_ref[...]
    out_ref[...] = (s + b_ref[...])[:, :n_classes]

  grid = (n // bm,)
  return pl.pallas_call(
      body,
      grid=grid,
      in_specs=[
          pl.BlockSpec((NC, bm, width), lambda i: (0, i, 0)),
          pl.BlockSpec((bm, 1), lambda i: (i, 0)),
          pl.BlockSpec((1, width), lambda i: (0, 0)),
      ],
      out_specs=pl.BlockSpec((bm, n_classes), lambda i: (i, 0)),
      out_shape=jax.ShapeDtypeStruct((n, n_classes), jnp.float32),
  )(aggp, dinv, b2p)


def kernel(x, edge_index, W1, b1, W2, b2):
  n, d_feat = x.shape
  hidden = W1.shape[1]
  n_classes = W2.shape[1]
  e = edge_index.shape[1]

  nbuf1, nbuf2 = 5, 5
  per = NW * CHUNK
  n2 = -(-(-(-e // per)) // 10) * 10
  e_pad = n2 * per
  n1 = 2 * n2
  ns1, ns2 = n1 // nbuf1, n2 // nbuf2
  ei = jnp.pad(edge_index.astype(jnp.int32), ((0, 0), (0, e_pad - e)),
               constant_values=DUMMY)
  src_f = ei[0]
  dst_f = ei[1]
  src_c = src_f.reshape(NS, ns1, nbuf1, CHUNK)
  dst_c = dst_f.reshape(NS, ns1, nbuf1, CHUNK)
  src_p = src_f.reshape(NW, ns2, nbuf2, CHUNK)
  dst_p = dst_f.reshape(NW, ns2, nbuf2, CHUNK)
  dst_d = dst_f.reshape(NW, n2, CHUNK)

  x_p = jnp.pad(x, ((0, N_PAD - n), (0, 0)))
  w2_width = -(-n_classes // 16) * 16
  w2_p = jnp.pad(W2, ((0, 0), (0, w2_width - n_classes)))
  b1_r = b1.reshape(1, hidden)
  b2_r = jnp.pad(b2, (0, w2_width - n_classes)).reshape(1, w2_width)

  zeros1 = jnp.zeros((N_PAD,), jnp.float32)
  zeros_w = jnp.zeros((N_PAD, w2_width), jnp.float32)

  degp = _make_deg(n2)(dst_d, zeros1)
  p1, dinv = _tc_first(x_p, W1, degp[..., None])
  agg1 = _make_agg_cols(ns1, hidden // 2, nbuf1)(p1, src_c, dst_c)
  p2 = _tc_mid(agg1, dinv, b1_r, w2_p)
  agg2 = _make_agg_edges(ns2, w2_width, nbuf2)(p2, src_p, dst_p, zeros_w)
  if n % 2000 == 0:
    return _tc_last(agg2, dinv, b2_r, n, n_classes)
  out = _tc_last(agg2, dinv, b2_r, N_PAD, w2_width)
  return out[:n, :n_classes]
